# baseline (speedup 1.0000x reference)
import jax
import jax.numpy as jnp
from jax import lax
from jax.experimental import pallas as pl
from jax.experimental.pallas import tpu as pltpu

_BIN_RATE = 0.5
_THRESHOLD = 0.0
_CHUNK = 256
_NBUF = 6


def _ring_body(nchunk, nbuf, mask_ref, x_hbm, o_hbm, ibuf, obuf, isem, osem):
    m = mask_ref[...]

    def in_copy(i, s):
        return pltpu.make_async_copy(
            x_hbm.at[pl.ds(i * _CHUNK, _CHUNK), :], ibuf.at[s], isem.at[s])

    def out_copy(i, s):
        return pltpu.make_async_copy(
            obuf.at[s], o_hbm.at[pl.ds(i * _CHUNK, _CHUNK), :], osem.at[s])

    for b in range(nbuf):
        in_copy(b, b).start()

    def step(i, carry):
        s = lax.rem(i, nbuf)
        in_copy(i, s).wait()

        @pl.when(i >= nbuf)
        def _():
            out_copy(i - nbuf, s).wait()

        x = ibuf[s]
        obuf[s] = jnp.where(m > 0.5, (x > _THRESHOLD).astype(jnp.float32),
                            jnp.maximum(x, 0.0))
        out_copy(i, s).start()

        @pl.when(i + nbuf < nchunk)
        def _():
            in_copy(i + nbuf, s).start()

        return carry

    lax.fori_loop(0, nchunk, step, 0)

    def drain(j, carry):
        i = nchunk - nbuf + j
        s = lax.rem(i, nbuf)
        out_copy(i, s).wait()
        return carry

    lax.fori_loop(0, nbuf, drain, 0)


def kernel(input):
    M, N = input.shape
    mask = jax.random.bernoulli(jax.random.key(42), _BIN_RATE, (N,))
    mask2 = mask.astype(jnp.float32)[None, :]
    nchunk = M // _CHUNK
    nbuf = min(_NBUF, nchunk)

    import functools
    body = functools.partial(_ring_body, nchunk, nbuf)

    return pl.pallas_call(
        body,
        in_specs=[
            pl.BlockSpec(memory_space=pltpu.VMEM),
            pl.BlockSpec(memory_space=pl.ANY),
        ],
        out_specs=pl.BlockSpec(memory_space=pl.ANY),
        out_shape=jax.ShapeDtypeStruct((M, N), jnp.float32),
        scratch_shapes=[
            pltpu.VMEM((nbuf, _CHUNK, N), jnp.float32),
            pltpu.VMEM((nbuf, _CHUNK, N), jnp.float32),
            pltpu.SemaphoreType.DMA((nbuf,)),
            pltpu.SemaphoreType.DMA((nbuf,)),
        ],
        compiler_params=pltpu.CompilerParams(
            vmem_limit_bytes=100 * 1024 * 1024,
        ),
    )(mask2, input)


# speedup vs baseline: 1.0110x; 1.0110x over previous
import jax
import jax.numpy as jnp
from jax.experimental import pallas as pl
from jax.experimental.pallas import tpu as pltpu

_BIN_RATE = 0.5
_THRESHOLD = 0.0


def _bin_dropout_block(mask_ref, x_ref, o_ref):
    x = x_ref[...]
    m = mask_ref[...]
    pos = x > _THRESHOLD
    o_ref[...] = jnp.where(m > 0.5, pos.astype(jnp.float32), jnp.maximum(x, 0.0))


def kernel(input):
    M, N = input.shape
    mask = jax.random.bernoulli(jax.random.key(42), _BIN_RATE, (N,))
    mask2 = mask.astype(jnp.float32)[None, :]
    block_rows = 1008
    grid = (pl.cdiv(M, block_rows),)
    return pl.pallas_call(
        _bin_dropout_block,
        grid=grid,
        in_specs=[
            pl.BlockSpec((1, N), lambda i: (0, 0)),
            pl.BlockSpec((block_rows, N), lambda i: (i, 0)),
        ],
        out_specs=pl.BlockSpec((block_rows, N), lambda i: (i, 0)),
        out_shape=jax.ShapeDtypeStruct((M, N), jnp.float32),
        compiler_params=pltpu.CompilerParams(
            dimension_semantics=("arbitrary",),
            vmem_limit_bytes=100 * 1024 * 1024,
        ),
    )(mask2, input)
